# Initial kernel scaffold; baseline (speedup 1.0000x reference)
import jax
import jax.numpy as jnp
from jax.experimental import pallas as pl


def kernel(z, embedding):
    raise NotImplementedError("write your pallas kernel here")


# speedup vs baseline: 1.1036x; 1.1036x over previous
import functools

import jax
import jax.numpy as jnp
from jax import lax
from jax.experimental import pallas as pl
from jax.experimental.pallas import tpu as pltpu
from jax.experimental.pallas import tpu_sc as plsc

NUM_CODES = 8192
EMBED_DIM = 256
COMMITMENT_COST = 0.25

_BR = 256


def _argmin_body(z_ref, e_ref, zn_ref, en_ref, idx_ref):
    z = z_ref[...]
    e = e_ref[...]
    zn = zn_ref[...]
    en = en_ref[...]
    mm = lax.dot_general(z, e, (((1,), (1,)), ((), ())),
                         preferred_element_type=jnp.float32)
    dist = (zn + en) - 2.0 * mm
    m = jnp.min(dist, axis=1, keepdims=True)
    ii = lax.broadcasted_iota(jnp.int32, dist.shape, 1)
    idx = jnp.min(jnp.where(dist == m, ii, jnp.int32(NUM_CODES)), axis=1)
    idx_ref[0, 0, :] = idx


def _compute_indices(z_flat, embedding, z_norm, e_norm):
    n = z_flat.shape[0]
    nb = n // _BR
    idx3 = pl.pallas_call(
        _argmin_body,
        grid=(nb,),
        in_specs=[
            pl.BlockSpec((_BR, EMBED_DIM), lambda i: (i, 0)),
            pl.BlockSpec((NUM_CODES, EMBED_DIM), lambda i: (0, 0)),
            pl.BlockSpec((_BR, 1), lambda i: (i, 0)),
            pl.BlockSpec((1, NUM_CODES), lambda i: (0, 0)),
        ],
        out_specs=pl.BlockSpec((1, 1, _BR), lambda i: (i, 0, 0)),
        out_shape=jax.ShapeDtypeStruct((nb, 1, _BR), jnp.int32),
    )(z_flat, embedding, z_norm, e_norm)
    return idx3.reshape(n)


_NC, _NS = 2, 16
_NW = _NC * _NS
_CH = 128


def _gather_count(indices, embedding):
    n = indices.shape[0]
    b_per_w = n // _NW
    n_chunks = b_per_w // _CH
    mesh = plsc.VectorSubcoreMesh(core_axis_name="c", subcore_axis_name="s")

    @functools.partial(
        pl.kernel,
        out_type=(
            jax.ShapeDtypeStruct((n, EMBED_DIM), jnp.float32),
            jax.ShapeDtypeStruct((_NW, NUM_CODES), jnp.float32),
        ),
        mesh=mesh,
        compiler_params=pltpu.CompilerParams(needs_layout_passes=False),
        scratch_types=[
            pltpu.VMEM((_CH,), jnp.int32),
            pltpu.VMEM((_CH, EMBED_DIM), jnp.float32),
            pltpu.VMEM((NUM_CODES,), jnp.float32),
            pltpu.SemaphoreType.DMA,
        ],
    )
    def k(idx_hbm, table_hbm, zq_hbm, counts_hbm, idx_v, rows_v, counts_v, sem):
        wid = lax.axis_index("s") * _NC + lax.axis_index("c")
        base = wid * b_per_w

        def zero_body(j, _):
            counts_v[pl.ds(j * 16, 16)] = jnp.zeros((16,), jnp.float32)
            return 0
        lax.fori_loop(0, NUM_CODES // 16, zero_body, 0)

        ones = jnp.ones((16,), jnp.float32)

        def chunk_body(c, _):
            off = base + c * _CH
            pltpu.sync_copy(idx_hbm.at[pl.ds(off, _CH)], idx_v)
            pltpu.async_copy(table_hbm.at[idx_v], rows_v, sem).wait()
            pltpu.sync_copy(rows_v, zq_hbm.at[pl.ds(off, _CH)])

            def cnt_body(j, _):
                v = idx_v[pl.ds(j * 16, 16)]
                plsc.addupdate_scatter(counts_v, [v], ones)
                return 0
            lax.fori_loop(0, _CH // 16, cnt_body, 0)
            return 0

        lax.fori_loop(0, n_chunks, chunk_body, 0)
        pltpu.sync_copy(counts_v, counts_hbm.at[wid])

    return k(indices, embedding)


def _epilogue_body(z_ref, zq_ref, cnt_ref, zqst_ref, res_ref, loss_ref,
                   perp_ref, acc_ref):
    i = pl.program_id(0)
    nb = pl.num_programs(0)

    @pl.when(i == 0)
    def _():
        acc_ref[0] = jnp.float32(0.0)

    z = z_ref[...]
    zq = zq_ref[...]
    d = zq - z
    zqst = z + d
    zqst_ref[...] = zqst
    res_ref[...] = z - zqst
    acc_ref[0] += jnp.sum(d * d)

    @pl.when(i == nb - 1)
    def _():
        n_total = nb * z_ref.shape[0] * z_ref.shape[1]
        loss = acc_ref[0] / n_total * COMMITMENT_COST
        loss_ref[...] = loss[None, None]
        counts = jnp.sum(cnt_ref[...], axis=0)
        avg = counts / (nb * z_ref.shape[0])
        ent = jnp.sum(avg * jnp.log(avg + 1e-10))
        perp_ref[...] = jnp.exp(-ent)[None, None]


def _epilogue(z_flat, z_q, counts):
    n = z_flat.shape[0]
    nb = n // _BR
    zqst, res, loss, perp = pl.pallas_call(
        _epilogue_body,
        grid=(nb,),
        in_specs=[
            pl.BlockSpec((_BR, EMBED_DIM), lambda i: (i, 0)),
            pl.BlockSpec((_BR, EMBED_DIM), lambda i: (i, 0)),
            pl.BlockSpec((_NW, NUM_CODES), lambda i: (0, 0)),
        ],
        out_specs=[
            pl.BlockSpec((_BR, EMBED_DIM), lambda i: (i, 0)),
            pl.BlockSpec((_BR, EMBED_DIM), lambda i: (i, 0)),
            pl.BlockSpec((1, 1), lambda i: (0, 0)),
            pl.BlockSpec((1, 1), lambda i: (0, 0)),
        ],
        out_shape=[
            jax.ShapeDtypeStruct((n, EMBED_DIM), jnp.float32),
            jax.ShapeDtypeStruct((n, EMBED_DIM), jnp.float32),
            jax.ShapeDtypeStruct((1, 1), jnp.float32),
            jax.ShapeDtypeStruct((1, 1), jnp.float32),
        ],
        scratch_shapes=[pltpu.SMEM((1,), jnp.float32)],
    )(z_flat, z_q, counts)
    return zqst, res, loss.reshape(()), perp.reshape(())


def kernel(z, embedding):
    z_flat = z.reshape(-1, EMBED_DIM)
    z_norm = jnp.sum(z_flat ** 2, axis=1, keepdims=True)
    e_norm = jnp.sum(embedding ** 2, axis=1).reshape(1, NUM_CODES)
    indices = _compute_indices(z_flat, embedding, z_norm, e_norm)
    z_q, counts = _gather_count(indices, embedding)
    zqst, res, loss, perp = _epilogue(z_flat, z_q, counts)
    return (zqst.reshape(z.shape), res.reshape(z.shape), indices, loss, perp)
